# baseline (speedup 1.0000x reference)
import functools

import jax
import jax.numpy as jnp
from jax import lax
from jax.experimental import pallas as pl
from jax.experimental.pallas import tpu as pltpu, tpu_sc as plsc

_N, _C, _H, _W = 1024, 256, 28, 28
_P = _H * _W
_NS = 2


def _mean_body(zt_ref, out_ref):
    w = None
    for j in range(4):
        for i in range(4):
            t = zt_ref[i, j]
            w = t if w is None else w + t
    wi, wj = pl.program_id(1), pl.program_id(2)
    first = (wi == 0) & (wj == 0)
    last = (wi == 6) & (wj == 6)

    @pl.when(first)
    def _init():
        out_ref[...] = w

    @pl.when(~first & ~last)
    def _accum():
        out_ref[...] = out_ref[...] + w

    @pl.when(last)
    def _final():
        out_ref[...] = (out_ref[...] + w) * jnp.float32(1.0 / _P)


def _spatial_mean(z):
    zt = jnp.transpose(z, (2, 3, 0, 1))
    nb = _N // _NS
    return pl.pallas_call(
        _mean_body,
        grid=(_NS, 7, 7),
        in_specs=[pl.BlockSpec((4, 4, nb, _C), lambda n, a, b: (a, b, n, 0))],
        out_specs=pl.BlockSpec((nb, _C), lambda n, a, b: (n, 0)),
        out_shape=jax.ShapeDtypeStruct((_N, _C), jnp.float32),
        compiler_params=pltpu.CompilerParams(
            dimension_semantics=("arbitrary", "arbitrary", "arbitrary")),
    )(zt)


def _sc_gate(yt):
    mesh = plsc.VectorSubcoreMesh(core_axis_name="c", subcore_axis_name="s")

    @functools.partial(
        pl.kernel, mesh=mesh,
        out_type=[jax.ShapeDtypeStruct((6, _N), jnp.float32),
                  jax.ShapeDtypeStruct((6, _N), jnp.float32)],
        scratch_types=[pltpu.VMEM((7, 32), jnp.float32),
                       pltpu.VMEM((6, 32), jnp.float32),
                       pltpu.VMEM((6, 32), jnp.float32)],
    )
    def gate(y_hbm, hard_hbm, soft_hbm, yv, hv, sv):
        wid = lax.axis_index("s") * 2 + lax.axis_index("c")
        base = wid * 32
        for k in range(7):
            pltpu.sync_copy(y_hbm.at[k, pl.ds(base, 32)], yv.at[k])
        for half in range(2):
            sl = pl.ds(half * 16, 16)
            y = [yv[k, sl] for k in range(7)]
            best = y[0]
            idx = jnp.zeros((16,), jnp.int32)
            for k in range(1, 7):
                p = y[k] > best
                best = jnp.where(p, y[k], best)
                idx = jnp.where(p, k, idx)
            m = y[0]
            for k in range(1, 7):
                m = jnp.maximum(m, y[k])
            e = [jnp.exp(yk - m) for yk in y]
            ssum = e[0]
            for k in range(1, 7):
                ssum = ssum + e[k]
            soft = [ek / ssum for ek in e]
            suf = soft[6]
            sv[5, sl] = suf
            for j in range(4, -1, -1):
                suf = suf + soft[j + 1]
                sv[j, sl] = suf
            for j in range(6):
                hv[j, sl] = jnp.where(idx >= j + 1, 1.0, 0.0).astype(jnp.float32)
        for j in range(6):
            pltpu.sync_copy(hv.at[j], hard_hbm.at[j, pl.ds(base, 32)])
            pltpu.sync_copy(sv.at[j], soft_hbm.at[j, pl.ds(base, 32)])

    return gate(yt)


def _bn_train(x, gamma, beta, eps=1e-5):
    mu = x.mean(0)
    var = x.var(0)
    return gamma * (x - mu) / jnp.sqrt(var + eps) + beta


def kernel(z, SNR, W1, b1, g1, be1, W2, b2, g2, be2, W3, b3, temp):
    feat = jnp.concatenate([_spatial_mean(z), SNR], axis=-1)
    h = feat @ W1.T + b1
    h = jax.nn.relu(h)
    h = _bn_train(h, g1, be1)
    h = h @ W2.T + b2
    h = jax.nn.relu(h)
    h = _bn_train(h, g2, be2)
    logits = h @ W3.T + b3
    g = jax.random.gumbel(jax.random.key(42), logits.shape, dtype=logits.dtype)
    y = (logits + g) / temp
    hard_t, soft_t = _sc_gate(y.T)
    return (hard_t.T, soft_t.T, logits)


# speedup vs baseline: 1.0020x; 1.0020x over previous
import functools

import jax
import jax.numpy as jnp
from jax import lax
from jax.experimental import pallas as pl
from jax.experimental.pallas import tpu as pltpu, tpu_sc as plsc

_N, _C, _H, _W = 1024, 256, 28, 28
_P = _H * _W
_NS = 2


def _mean_body(zt_ref, out_ref):
    w = None
    for j in range(4):
        for i in range(4):
            t = zt_ref[i, j]
            w = t if w is None else w + t
    wi, wj = pl.program_id(1), pl.program_id(2)
    first = (wi == 0) & (wj == 0)
    last = (wi == 6) & (wj == 6)

    @pl.when(first)
    def _init():
        out_ref[...] = w

    @pl.when(~first & ~last)
    def _accum():
        out_ref[...] = out_ref[...] + w

    @pl.when(last)
    def _final():
        out_ref[...] = (out_ref[...] + w) * jnp.float32(1.0 / _P)


def _spatial_mean(z):
    zt = jnp.transpose(z, (2, 3, 0, 1))
    nb = _N // _NS
    return pl.pallas_call(
        _mean_body,
        grid=(_NS, 7, 7),
        in_specs=[pl.BlockSpec((4, 4, nb, _C), lambda n, a, b: (a, b, n, 0))],
        out_specs=pl.BlockSpec((nb, _C), lambda n, a, b: (n, 0)),
        out_shape=jax.ShapeDtypeStruct((_N, _C), jnp.float32),
        compiler_params=pltpu.CompilerParams(
            dimension_semantics=("arbitrary", "arbitrary", "arbitrary")),
    )(zt)


def _sc_gate(yt):
    mesh = plsc.VectorSubcoreMesh(core_axis_name="c", subcore_axis_name="s")

    @functools.partial(
        pl.kernel, mesh=mesh,
        out_type=[jax.ShapeDtypeStruct((6, _N), jnp.float32),
                  jax.ShapeDtypeStruct((6, _N), jnp.float32)],
        scratch_types=[pltpu.VMEM((7 * _N,), jnp.float32),
                       pltpu.VMEM((6, 32), jnp.float32),
                       pltpu.VMEM((6, 32), jnp.float32),
                       pltpu.SemaphoreType.DMA],
    )
    def gate(y_hbm, hard_hbm, soft_hbm, yv, hv, sv, sem):
        wid = lax.axis_index("s") * 2 + lax.axis_index("c")
        base = wid * 32
        pltpu.sync_copy(y_hbm, yv)
        for half in range(2):
            sl = pl.ds(half * 16, 16)
            y = [yv[pl.ds(k * _N + base + half * 16, 16)] for k in range(7)]
            best = y[0]
            idx = jnp.zeros((16,), jnp.int32)
            for k in range(1, 7):
                p = y[k] > best
                best = jnp.where(p, y[k], best)
                idx = jnp.where(p, k, idx)
            m = y[0]
            for k in range(1, 7):
                m = jnp.maximum(m, y[k])
            e = [jnp.exp(yk - m) for yk in y]
            ssum = e[0]
            for k in range(1, 7):
                ssum = ssum + e[k]
            soft = [ek / ssum for ek in e]
            suf = soft[6]
            sv[5, sl] = suf
            for j in range(4, -1, -1):
                suf = suf + soft[j + 1]
                sv[j, sl] = suf
            for j in range(6):
                hv[j, sl] = jnp.where(idx >= j + 1, 1.0, 0.0).astype(jnp.float32)
        copies = []
        for j in range(6):
            copies.append(pltpu.async_copy(
                hv.at[j], hard_hbm.at[j, pl.ds(base, 32)], sem))
            copies.append(pltpu.async_copy(
                sv.at[j], soft_hbm.at[j, pl.ds(base, 32)], sem))
        for c in copies:
            c.wait()

    return gate(yt.reshape(7 * _N))


def _bn_train(x, gamma, beta, eps=1e-5):
    mu = x.mean(0)
    var = x.var(0)
    return gamma * (x - mu) / jnp.sqrt(var + eps) + beta


def kernel(z, SNR, W1, b1, g1, be1, W2, b2, g2, be2, W3, b3, temp):
    feat = jnp.concatenate([_spatial_mean(z), SNR], axis=-1)
    h = feat @ W1.T + b1
    h = jax.nn.relu(h)
    h = _bn_train(h, g1, be1)
    h = h @ W2.T + b2
    h = jax.nn.relu(h)
    h = _bn_train(h, g2, be2)
    logits = h @ W3.T + b3
    g = jax.random.gumbel(jax.random.key(42), logits.shape, dtype=logits.dtype)
    y = (logits + g) / temp
    hard_t, soft_t = _sc_gate(y.T)
    return (hard_t.T, soft_t.T, logits)
